# baseline (speedup 1.0000x reference)
import functools

import jax
import jax.numpy as jnp
from jax import lax
from jax.experimental import pallas as pl
from jax.experimental.pallas import tpu as pltpu
from jax.experimental.pallas import tpu_sc as plsc

B, L, V, H = 4, 2048, 30522, 768
EPS = 1e-12

NC, NS = 2, 16
NW = NC * NS
L_PER_W = L // NW
P = 8
CHUNK = B * P
N_CHUNK = L_PER_W // P
NBUF = 3
LANES = 16
NV = H // LANES


def _rsqrt(x):
    i = lax.bitcast_convert_type(x, jnp.int32)
    y = lax.bitcast_convert_type(jnp.int32(0x5F3759DF) - (i >> 1), jnp.float32)
    for _ in range(3):
        y = y * (1.5 - 0.5 * x * y * y)
    return y


def _body(ids_hbm, pos_hbm, tab_hbm, w_hbm, bias_hbm, out_hbm,
          idx_v, pos_v, g0, g1, g2, w_v, bias_v, m_s, i_s,
          gs0, gs1, gs2, ss0, ss1, ss2):
    cid = lax.axis_index("c")
    sid = lax.axis_index("s")
    wid = sid * NC + cid
    l0 = wid * L_PER_W

    bufs = (g0, g1, g2)
    gsems = (gs0, gs1, gs2)
    ssems = (ss0, ss1, ss2)

    pltpu.sync_copy(ids_hbm.at[wid], idx_v)

    gathers = [None] * N_CHUNK
    stores = [None] * N_CHUNK
    gathers[0] = pltpu.async_copy(tab_hbm.at[idx_v.at[0]], bufs[0], gsems[0])
    gathers[1] = pltpu.async_copy(tab_hbm.at[idx_v.at[1]], bufs[1], gsems[1])

    pltpu.sync_copy(pos_hbm.at[pl.ds(l0, L_PER_W)], pos_v)
    pltpu.sync_copy(w_hbm, w_v)
    pltpu.sync_copy(bias_hbm, bias_v)

    def compute(gbuf, j):
        def sum_body(p, carry):
            pr = j * P + p

            z = jnp.zeros((LANES,), jnp.float32)

            @plsc.parallel_loop(0, NV, 2, unroll=3,
                                carry=(z, z, z, z, z, z, z, z))
            def slice_body(k, accs):
                a0, a1, a2, a3, q0, q1, q2, q3 = accs
                for dk in range(2):
                    sl = pl.ds((k + dk) * LANES, LANES)
                    pv = pos_v[pr, sl]
                    v0 = gbuf[p, sl] + pv
                    v1 = gbuf[P + p, sl] + pv
                    v2 = gbuf[2 * P + p, sl] + pv
                    v3 = gbuf[3 * P + p, sl] + pv
                    gbuf[p, sl] = v0
                    gbuf[P + p, sl] = v1
                    gbuf[2 * P + p, sl] = v2
                    gbuf[3 * P + p, sl] = v3
                    a0, q0 = a0 + v0, q0 + v0 * v0
                    a1, q1 = a1 + v1, q1 + v1 * v1
                    a2, q2 = a2 + v2, q2 + v2 * v2
                    a3, q3 = a3 + v3, q3 + v3 * v3
                return a0, a1, a2, a3, q0, q1, q2, q3

            a0, a1, a2, a3, q0, q1, q2, q3 = slice_body
            for b, (av, qv) in enumerate(((a0, q0), (a1, q1),
                                          (a2, q2), (a3, q3))):
                mean = jnp.sum(av) * (1.0 / H)
                var = jnp.maximum(jnp.sum(qv) * (1.0 / H) - mean * mean, 0.0)
                r = b * P + p
                m_s[r] = mean
                i_s[r] = _rsqrt(var + EPS)
            return carry

        lax.fori_loop(0, P, sum_body, 0)

        def norm_slice(k, carry):
            sl = pl.ds(k * LANES, LANES)
            wv = w_v[sl]
            bv = bias_v[sl]

            def norm_rows(rg, c2):
                r0 = rg * 8
                for dr in range(8):
                    r = r0 + dr
                    gbuf[r, sl] = (gbuf[r, sl] - m_s[r]) * i_s[r] * wv + bv
                return c2

            lax.fori_loop(0, CHUNK // 8, norm_rows, 0)
            return carry

        lax.fori_loop(0, NV, norm_slice, 0)

    for j in range(N_CHUNK):
        bi = j % NBUF
        gathers[j].wait()
        compute(bufs[bi], j)
        stores[j] = tuple(
            pltpu.async_copy(bufs[bi].at[pl.ds(b * P, P)],
                             out_hbm.at[pl.ds(b * L + l0 + j * P, P)],
                             ssems[bi])
            for b in range(B))
        nj = j + 2
        if nj < N_CHUNK:
            nbi = nj % NBUF
            if nj - NBUF >= 0:
                for st in stores[nj - NBUF]:
                    st.wait()
            gathers[nj] = pltpu.async_copy(
                tab_hbm.at[idx_v.at[nj]], bufs[nbi], gsems[nbi])
    for j in range(N_CHUNK - NBUF, N_CHUNK):
        for st in stores[j]:
            st.wait()


def kernel(input_ids, attention_mask, word_embeddings, position_embeddings,
           ln_weight, ln_bias):
    del attention_mask
    ids_perm = (input_ids.astype(jnp.int32)
                .reshape(B, NW, N_CHUNK, P)
                .transpose(1, 2, 0, 3)
                .reshape(NW, N_CHUNK, CHUNK))
    mesh = plsc.VectorSubcoreMesh(
        core_axis_name="c", subcore_axis_name="s",
        num_cores=NC, num_subcores=NS)
    fn = functools.partial(
        pl.kernel,
        out_type=jax.ShapeDtypeStruct((B * L, H), jnp.float32),
        mesh=mesh,
        scratch_types=[
            pltpu.VMEM((N_CHUNK, CHUNK), jnp.int32),
            pltpu.VMEM((L_PER_W, H), jnp.float32),
            pltpu.VMEM((CHUNK, H), jnp.float32),
            pltpu.VMEM((CHUNK, H), jnp.float32),
            pltpu.VMEM((CHUNK, H), jnp.float32),
            pltpu.VMEM((H,), jnp.float32),
            pltpu.VMEM((H,), jnp.float32),
            pltpu.SMEM((CHUNK,), jnp.float32),
            pltpu.SMEM((CHUNK,), jnp.float32),
            pltpu.SemaphoreType.DMA,
            pltpu.SemaphoreType.DMA,
            pltpu.SemaphoreType.DMA,
            pltpu.SemaphoreType.DMA,
            pltpu.SemaphoreType.DMA,
            pltpu.SemaphoreType.DMA,
        ],
        compiler_params=pltpu.CompilerParams(needs_layout_passes=False),
    )(_body)
    out = fn(ids_perm, position_embeddings, word_embeddings, ln_weight, ln_bias)
    return out.reshape(B, L, H)


# speedup vs baseline: 1.0070x; 1.0070x over previous
import functools

import jax
import jax.numpy as jnp
from jax import lax
from jax.experimental import pallas as pl
from jax.experimental.pallas import tpu as pltpu
from jax.experimental.pallas import tpu_sc as plsc

B, L, V, H = 4, 2048, 30522, 768
EPS = 1e-12

NC, NS = 2, 16
NW = NC * NS
L_PER_W = L // NW
P = 8
CHUNK = B * P
N_CHUNK = L_PER_W // P
NBUF = 3
LANES = 16
NV = H // LANES


def _rsqrt(x):
    i = lax.bitcast_convert_type(x, jnp.int32)
    y = lax.bitcast_convert_type(jnp.int32(0x5F3759DF) - (i >> 1), jnp.float32)
    for _ in range(3):
        y = y * (1.5 - 0.5 * x * y * y)
    return y


def _body(ids_hbm, pos_hbm, tab_hbm, w_hbm, bias_hbm, out_hbm,
          idx_v, pos_v, g0, g1, g2, w_v, bias_v, m_s, i_s,
          gs0, gs1, gs2, ss0, ss1, ss2):
    cid = lax.axis_index("c")
    sid = lax.axis_index("s")
    wid = sid * NC + cid
    l0 = wid * L_PER_W

    bufs = (g0, g1, g2)
    gsems = (gs0, gs1, gs2)
    ssems = (ss0, ss1, ss2)

    pltpu.sync_copy(ids_hbm.at[wid], idx_v)

    gathers = [None] * N_CHUNK
    stores = [None] * N_CHUNK
    gathers[0] = pltpu.async_copy(tab_hbm.at[idx_v.at[0]], bufs[0], gsems[0])
    gathers[1] = pltpu.async_copy(tab_hbm.at[idx_v.at[1]], bufs[1], gsems[1])

    pltpu.sync_copy(pos_hbm.at[pl.ds(l0, L_PER_W)], pos_v)
    pltpu.sync_copy(w_hbm, w_v)
    pltpu.sync_copy(bias_hbm, bias_v)

    def compute(gbuf, j):
        def sum_body(p, carry):
            pr = j * P + p

            z = jnp.zeros((LANES,), jnp.float32)

            @plsc.parallel_loop(0, NV, 2, unroll=2,
                                carry=(z, z, z, z, z, z, z, z))
            def slice_body(k, accs):
                a0, a1, a2, a3, q0, q1, q2, q3 = accs
                for dk in range(2):
                    sl = pl.ds((k + dk) * LANES, LANES)
                    pv = pos_v[pr, sl]
                    v0 = gbuf[p, sl] + pv
                    v1 = gbuf[P + p, sl] + pv
                    v2 = gbuf[2 * P + p, sl] + pv
                    v3 = gbuf[3 * P + p, sl] + pv
                    gbuf[p, sl] = v0
                    gbuf[P + p, sl] = v1
                    gbuf[2 * P + p, sl] = v2
                    gbuf[3 * P + p, sl] = v3
                    a0, q0 = a0 + v0, q0 + v0 * v0
                    a1, q1 = a1 + v1, q1 + v1 * v1
                    a2, q2 = a2 + v2, q2 + v2 * v2
                    a3, q3 = a3 + v3, q3 + v3 * v3
                return a0, a1, a2, a3, q0, q1, q2, q3

            a0, a1, a2, a3, q0, q1, q2, q3 = slice_body
            for b, (av, qv) in enumerate(((a0, q0), (a1, q1),
                                          (a2, q2), (a3, q3))):
                mean = jnp.sum(av) * (1.0 / H)
                var = jnp.maximum(jnp.sum(qv) * (1.0 / H) - mean * mean, 0.0)
                r = b * P + p
                m_s[r] = mean
                i_s[r] = _rsqrt(var + EPS)
            return carry

        lax.fori_loop(0, P, sum_body, 0)

        def norm_slice(k, carry):
            sl = pl.ds(k * LANES, LANES)
            wv = w_v[sl]
            bv = bias_v[sl]

            def norm_rows(rg, c2):
                r0 = rg * 8
                for dr in range(8):
                    r = r0 + dr
                    gbuf[r, sl] = (gbuf[r, sl] - m_s[r]) * i_s[r] * wv + bv
                return c2

            lax.fori_loop(0, CHUNK // 8, norm_rows, 0)
            return carry

        lax.fori_loop(0, NV, norm_slice, 0)

    for j in range(N_CHUNK):
        bi = j % NBUF
        gathers[j].wait()
        compute(bufs[bi], j)
        stores[j] = tuple(
            pltpu.async_copy(bufs[bi].at[pl.ds(b * P, P)],
                             out_hbm.at[pl.ds(b * L + l0 + j * P, P)],
                             ssems[bi])
            for b in range(B))
        nj = j + 2
        if nj < N_CHUNK:
            nbi = nj % NBUF
            if nj - NBUF >= 0:
                for st in stores[nj - NBUF]:
                    st.wait()
            gathers[nj] = pltpu.async_copy(
                tab_hbm.at[idx_v.at[nj]], bufs[nbi], gsems[nbi])
    for j in range(N_CHUNK - NBUF, N_CHUNK):
        for st in stores[j]:
            st.wait()


def kernel(input_ids, attention_mask, word_embeddings, position_embeddings,
           ln_weight, ln_bias):
    del attention_mask
    ids_perm = (input_ids.astype(jnp.int32)
                .reshape(B, NW, N_CHUNK, P)
                .transpose(1, 2, 0, 3)
                .reshape(NW, N_CHUNK, CHUNK))
    mesh = plsc.VectorSubcoreMesh(
        core_axis_name="c", subcore_axis_name="s",
        num_cores=NC, num_subcores=NS)
    fn = functools.partial(
        pl.kernel,
        out_type=jax.ShapeDtypeStruct((B * L, H), jnp.float32),
        mesh=mesh,
        scratch_types=[
            pltpu.VMEM((N_CHUNK, CHUNK), jnp.int32),
            pltpu.VMEM((L_PER_W, H), jnp.float32),
            pltpu.VMEM((CHUNK, H), jnp.float32),
            pltpu.VMEM((CHUNK, H), jnp.float32),
            pltpu.VMEM((CHUNK, H), jnp.float32),
            pltpu.VMEM((H,), jnp.float32),
            pltpu.VMEM((H,), jnp.float32),
            pltpu.SMEM((CHUNK,), jnp.float32),
            pltpu.SMEM((CHUNK,), jnp.float32),
            pltpu.SemaphoreType.DMA,
            pltpu.SemaphoreType.DMA,
            pltpu.SemaphoreType.DMA,
            pltpu.SemaphoreType.DMA,
            pltpu.SemaphoreType.DMA,
            pltpu.SemaphoreType.DMA,
        ],
        compiler_params=pltpu.CompilerParams(needs_layout_passes=False),
    )(_body)
    out = fn(ids_perm, position_embeddings, word_embeddings, ln_weight, ln_bias)
    return out.reshape(B, L, H)
